# baseline (speedup 1.0000x reference)
import functools

import jax
import jax.numpy as jnp
from jax.experimental import pallas as pl
from jax.experimental.pallas import tpu as pltpu

K_TOP = 64


def _encode_kernel(x_ref, w_ref, b_ref, out_ref):
    out_ref[...] = (
        jnp.dot(x_ref[...], w_ref[...], preferred_element_type=jnp.float32)
        + b_ref[...]
    )


def _threshold_kernel(p_ref, thr_ref, *, k):
    p = p_ref[...]
    rows, width = p.shape
    kf = float(k)

    m = jnp.max(p.reshape(rows, 8, width // 8), axis=1)

    mum = jnp.mean(m, axis=1, keepdims=True)
    msq = jnp.mean(m * m, axis=1, keepdims=True)
    sigm = jnp.sqrt(jnp.maximum(msq - mum * mum, 1e-30))

    def newton_body(_, carry):
        t, lo, hi, best_c, tbest = carry
        cnt = jnp.sum((m >= t).astype(jnp.float32), axis=1, keepdims=True)
        take = (cnt >= kf) & (cnt < best_c) & (t > 0.0)
        best_c = jnp.where(take, cnt, best_c)
        tbest = jnp.where(take, t, tbest)
        tkey = jax.lax.bitcast_convert_type(t, jnp.int32)
        lo = jnp.where(cnt > kf, jnp.maximum(lo, tkey), lo)
        hi = jnp.where(cnt <= kf, jnp.minimum(hi, tkey), hi)
        arg = (t - mum) * (t - mum) + 2.0 * sigm * sigm * jnp.log(
            jnp.maximum(cnt, 0.5) / kf
        )
        tn = mum + jnp.sqrt(jnp.maximum(arg, 0.0))
        kn = jax.lax.bitcast_convert_type(tn, jnp.int32)
        bad = (kn <= lo) | (kn >= hi)
        kmid = lo + ((hi - lo) >> 1)
        kn = jnp.where(bad, kmid, kn)
        t = jax.lax.bitcast_convert_type(kn, jnp.float32)
        return t, lo, hi, best_c, tbest

    _, _, _, _, tbest = jax.lax.fori_loop(
        0,
        12,
        newton_body,
        (
            mum + 2.2 * sigm,
            jnp.zeros((rows, 1), jnp.int32),
            jnp.full((rows, 1), jnp.int32(0x7F800000)),
            jnp.full((rows, 1), jnp.float32(jnp.inf)),
            jnp.zeros((rows, 1), jnp.float32),
        ),
    )

    surv = jnp.where(p >= tbest, p, jnp.inf)
    cfull = jnp.sum((p >= tbest).astype(jnp.float32), axis=1, keepdims=True)
    nseg = 2048
    a = jnp.min(surv.reshape(rows, width // nseg, nseg), axis=1)
    extras = cfull - kf
    cand = tbest
    s = jnp.min(a, axis=1, keepdims=True)
    for i in range(6):
        nxt = jax.lax.bitcast_convert_type(
            jax.lax.bitcast_convert_type(jnp.abs(s), jnp.int32) + 1, jnp.float32
        )
        cand = jnp.where(extras == float(i + 1), nxt, cand)
        s = jnp.min(jnp.where(a > s, a, jnp.inf), axis=1, keepdims=True)
    cv = jnp.sum((p >= cand).astype(jnp.float32), axis=1, keepdims=True)
    need = (cv != kf) & (cfull > kf)
    t_in = jnp.where(need, tbest, cand)
    c_in = jnp.where(need, cfull, kf)

    def rem_cond(carry):
        return jnp.any(carry[1] > kf)

    def rem_body(carry):
        t, c = carry
        v = jnp.min(jnp.where(p >= t, p, jnp.inf), axis=1, keepdims=True)
        v = jnp.abs(v)
        mult = jnp.sum((p == v).astype(jnp.float32), axis=1, keepdims=True)
        candc = c - mult
        ok = candc >= kf
        nxt = jax.lax.bitcast_convert_type(
            jax.lax.bitcast_convert_type(v, jnp.int32) + 1, jnp.float32
        )
        active = c > kf
        t = jnp.where(active, jnp.where(ok, nxt, v), t)
        c = jnp.where(active, jnp.where(ok, candc, kf), c)
        return t, c

    t_fin, _ = jax.lax.while_loop(rem_cond, rem_body, (t_in, c_in))
    thr_ref[...] = t_fin


def _decode_kernel(p_ref, thr_ref, w_ref, b_ref, lat_ref, out_ref):
    lt = pl.program_id(1)
    p = p_ref[...]
    lat = jnp.where(p >= thr_ref[...], jnp.maximum(p, 0.0), 0.0)
    lat_ref[...] = lat
    contrib = jnp.dot(lat, w_ref[...], preferred_element_type=jnp.float32)

    @pl.when(lt == 0)
    def _():
        out_ref[...] = contrib + b_ref[...]

    @pl.when(lt != 0)
    def _():
        out_ref[...] += contrib


@jax.jit
def kernel(x, W_enc, b_enc, W_dec, b_dec):
    n, d = x.shape
    l = W_enc.shape[1]

    r1 = min(512, n)
    lt_size = min(2048, l)
    n_lt = l // lt_size
    n_nb = n // r1

    b_enc2 = b_enc.reshape(1, l)
    b_dec2 = b_dec.reshape(1, d)

    preact = pl.pallas_call(
        _encode_kernel,
        grid=(n_lt, n_nb),
        in_specs=[
            pl.BlockSpec((r1, d), lambda lt, nb: (nb, 0)),
            pl.BlockSpec((d, lt_size), lambda lt, nb: (0, lt)),
            pl.BlockSpec((1, lt_size), lambda lt, nb: (0, lt)),
        ],
        out_specs=pl.BlockSpec((r1, lt_size), lambda lt, nb: (nb, lt)),
        out_shape=jax.ShapeDtypeStruct((n, l), jnp.float32),
        compiler_params=pltpu.CompilerParams(
            dimension_semantics=("arbitrary", "arbitrary"),
        ),
    )(x, W_enc, b_enc2)

    r_thr = min(64, n)
    thresholds = pl.pallas_call(
        functools.partial(_threshold_kernel, k=K_TOP),
        grid=(n // r_thr,),
        in_specs=[pl.BlockSpec((r_thr, l), lambda i: (i, 0))],
        out_specs=pl.BlockSpec((r_thr, 1), lambda i: (i, 0)),
        out_shape=jax.ShapeDtypeStruct((n, 1), jnp.float32),
    )(preact)

    r2 = min(1024, n)
    latents, out = pl.pallas_call(
        _decode_kernel,
        grid=(n // r2, n_lt),
        in_specs=[
            pl.BlockSpec((r2, lt_size), lambda nb, lt: (nb, lt)),
            pl.BlockSpec((r2, 1), lambda nb, lt: (nb, 0)),
            pl.BlockSpec((lt_size, d), lambda nb, lt: (lt, 0)),
            pl.BlockSpec((1, d), lambda nb, lt: (0, 0)),
        ],
        out_specs=[
            pl.BlockSpec((r2, lt_size), lambda nb, lt: (nb, lt)),
            pl.BlockSpec((r2, d), lambda nb, lt: (nb, 0)),
        ],
        out_shape=[
            jax.ShapeDtypeStruct((n, l), jnp.float32),
            jax.ShapeDtypeStruct((n, d), jnp.float32),
        ],
        compiler_params=pltpu.CompilerParams(
            dimension_semantics=("parallel", "arbitrary"),
        ),
    )(preact, thresholds, W_dec, b_dec2)

    num_dead = jnp.array(0, dtype=jnp.int32)
    return (latents, out, preact, num_dead)


# speedup vs baseline: 1.6852x; 1.6852x over previous
import functools

import jax
import jax.numpy as jnp
from jax.experimental import pallas as pl
from jax.experimental.pallas import tpu as pltpu

K_TOP = 64


def _encode_kernel(x_ref, w_ref, b_ref, out_ref, m_ref):
    p = (
        jnp.dot(x_ref[...], w_ref[...], preferred_element_type=jnp.float32)
        + b_ref[...]
    )
    out_ref[...] = p
    rows, cols = p.shape
    m_ref[...] = jnp.max(p.reshape(rows, 8, cols // 8), axis=1)


def _newton_kernel(m_ref, tbest_ref, *, k):
    m = m_ref[...]
    rows = m.shape[0]
    kf = float(k)
    mum = jnp.mean(m, axis=1, keepdims=True)
    msq = jnp.mean(m * m, axis=1, keepdims=True)
    sigm = jnp.sqrt(jnp.maximum(msq - mum * mum, 1e-30))

    def newton_body(_, carry):
        t, lo, hi, best_c, tbest = carry
        cnt = jnp.sum((m >= t).astype(jnp.float32), axis=1, keepdims=True)
        take = (cnt >= kf) & (cnt < best_c) & (t > 0.0)
        best_c = jnp.where(take, cnt, best_c)
        tbest = jnp.where(take, t, tbest)
        tkey = jax.lax.bitcast_convert_type(t, jnp.int32)
        lo = jnp.where(cnt > kf, jnp.maximum(lo, tkey), lo)
        hi = jnp.where(cnt <= kf, jnp.minimum(hi, tkey), hi)
        arg = (t - mum) * (t - mum) + 2.0 * sigm * sigm * jnp.log(
            jnp.maximum(cnt, 0.5) / kf
        )
        tn = mum + jnp.sqrt(jnp.maximum(arg, 0.0))
        kn = jax.lax.bitcast_convert_type(tn, jnp.int32)
        bad = (kn <= lo) | (kn >= hi)
        kmid = lo + ((hi - lo) >> 1)
        kn = jnp.where(bad, kmid, kn)
        t = jax.lax.bitcast_convert_type(kn, jnp.float32)
        return t, lo, hi, best_c, tbest

    _, _, _, _, tbest = jax.lax.fori_loop(
        0,
        12,
        newton_body,
        (
            mum + 2.2 * sigm,
            jnp.zeros((rows, 1), jnp.int32),
            jnp.full((rows, 1), jnp.int32(0x7F800000)),
            jnp.full((rows, 1), jnp.float32(jnp.inf)),
            jnp.zeros((rows, 1), jnp.float32),
        ),
    )
    tbest_ref[...] = tbest


def _tail_kernel(p_ref, tb_ref, thr_ref, *, k):
    p = p_ref[...]
    tbest = tb_ref[...]
    rows, width = p.shape
    kf = float(k)

    surv = jnp.where(p >= tbest, p, jnp.inf)
    cfull = jnp.sum((p >= tbest).astype(jnp.float32), axis=1, keepdims=True)
    nseg = min(2048, width)
    a = jnp.min(surv.reshape(rows, width // nseg, nseg), axis=1)
    extras = cfull - kf
    cand = tbest
    s = jnp.min(a, axis=1, keepdims=True)
    for i in range(6):
        nxt = jax.lax.bitcast_convert_type(
            jax.lax.bitcast_convert_type(jnp.abs(s), jnp.int32) + 1, jnp.float32
        )
        cand = jnp.where(extras == float(i + 1), nxt, cand)
        s = jnp.min(jnp.where(a > s, a, jnp.inf), axis=1, keepdims=True)
    cv = jnp.sum((p >= cand).astype(jnp.float32), axis=1, keepdims=True)
    need = (cv != kf) & (cfull > kf)
    t_in = jnp.where(need, tbest, cand)
    c_in = jnp.where(need, cfull, kf)

    def rem_cond(carry):
        return jnp.any(carry[1] > kf)

    def rem_body(carry):
        t, c = carry
        v = jnp.min(jnp.where(p >= t, p, jnp.inf), axis=1, keepdims=True)
        v = jnp.abs(v)
        mult = jnp.sum((p == v).astype(jnp.float32), axis=1, keepdims=True)
        candc = c - mult
        ok = candc >= kf
        nxt = jax.lax.bitcast_convert_type(
            jax.lax.bitcast_convert_type(v, jnp.int32) + 1, jnp.float32
        )
        active = c > kf
        t = jnp.where(active, jnp.where(ok, nxt, v), t)
        c = jnp.where(active, jnp.where(ok, candc, kf), c)
        return t, c

    t_fin, _ = jax.lax.while_loop(rem_cond, rem_body, (t_in, c_in))
    thr_ref[...] = t_fin


def _decode_kernel(p_ref, thr_ref, w_ref, b_ref, lat_ref, out_ref):
    lt = pl.program_id(1)
    p = p_ref[...]
    lat = jnp.where(p >= thr_ref[...], jnp.maximum(p, 0.0), 0.0)
    lat_ref[...] = lat
    contrib = jnp.dot(lat, w_ref[...], preferred_element_type=jnp.float32)

    @pl.when(lt == 0)
    def _():
        out_ref[...] = contrib + b_ref[...]

    @pl.when(lt != 0)
    def _():
        out_ref[...] += contrib


@jax.jit
def kernel(x, W_enc, b_enc, W_dec, b_dec):
    n, d = x.shape
    l = W_enc.shape[1]
    lp = l // 8

    r1 = min(512, n)
    lt_size = min(2048, l)
    n_lt = l // lt_size
    n_nb = n // r1

    b_enc2 = b_enc.reshape(1, l)
    b_dec2 = b_dec.reshape(1, d)

    preact, pooled = pl.pallas_call(
        _encode_kernel,
        grid=(n_lt, n_nb),
        in_specs=[
            pl.BlockSpec((r1, d), lambda lt, nb: (nb, 0)),
            pl.BlockSpec((d, lt_size), lambda lt, nb: (0, lt)),
            pl.BlockSpec((1, lt_size), lambda lt, nb: (0, lt)),
        ],
        out_specs=[
            pl.BlockSpec((r1, lt_size), lambda lt, nb: (nb, lt)),
            pl.BlockSpec((r1, lt_size // 8), lambda lt, nb: (nb, lt)),
        ],
        out_shape=[
            jax.ShapeDtypeStruct((n, l), jnp.float32),
            jax.ShapeDtypeStruct((n, lp), jnp.float32),
        ],
        compiler_params=pltpu.CompilerParams(
            dimension_semantics=("arbitrary", "arbitrary"),
        ),
    )(x, W_enc, b_enc2)

    r_nw = min(512, n)
    tbest = pl.pallas_call(
        functools.partial(_newton_kernel, k=K_TOP),
        grid=(n // r_nw,),
        in_specs=[pl.BlockSpec((r_nw, lp), lambda i: (i, 0))],
        out_specs=pl.BlockSpec((r_nw, 1), lambda i: (i, 0)),
        out_shape=jax.ShapeDtypeStruct((n, 1), jnp.float32),
    )(pooled)

    r_thr = min(64, n)
    thresholds = pl.pallas_call(
        functools.partial(_tail_kernel, k=K_TOP),
        grid=(n // r_thr,),
        in_specs=[
            pl.BlockSpec((r_thr, l), lambda i: (i, 0)),
            pl.BlockSpec((r_thr, 1), lambda i: (i, 0)),
        ],
        out_specs=pl.BlockSpec((r_thr, 1), lambda i: (i, 0)),
        out_shape=jax.ShapeDtypeStruct((n, 1), jnp.float32),
    )(preact, tbest)

    r2 = min(1024, n)
    latents, out = pl.pallas_call(
        _decode_kernel,
        grid=(n // r2, n_lt),
        in_specs=[
            pl.BlockSpec((r2, lt_size), lambda nb, lt: (nb, lt)),
            pl.BlockSpec((r2, 1), lambda nb, lt: (nb, 0)),
            pl.BlockSpec((lt_size, d), lambda nb, lt: (lt, 0)),
            pl.BlockSpec((1, d), lambda nb, lt: (0, 0)),
        ],
        out_specs=[
            pl.BlockSpec((r2, lt_size), lambda nb, lt: (nb, lt)),
            pl.BlockSpec((r2, d), lambda nb, lt: (nb, 0)),
        ],
        out_shape=[
            jax.ShapeDtypeStruct((n, l), jnp.float32),
            jax.ShapeDtypeStruct((n, d), jnp.float32),
        ],
        compiler_params=pltpu.CompilerParams(
            dimension_semantics=("parallel", "arbitrary"),
        ),
    )(preact, thresholds, W_dec, b_dec2)

    num_dead = jnp.array(0, dtype=jnp.int32)
    return (latents, out, preact, num_dead)
